# baseline (speedup 1.0000x reference)
import functools

import jax
import jax.numpy as jnp
from jax import lax
from jax.experimental import pallas as pl
from jax.experimental.pallas import tpu as pltpu
from jax.experimental.pallas import tpu_sc as plsc

N_ROWS = 16384
NUM_Q = 1024
VEC_DIM = 64
ROW_BLOCK = 2048


def _norm_over_rows(t):
    a = t[0:8, :]
    for c in range(1, 8):
        a = a + t[8 * c:8 * c + 8, :]
    for width in (4, 2, 1):
        a = a[:width, :] + a[width:, :]
    return a


def _norm_over_cols(t):
    a = t[:, 0:8]
    for c in range(1, 8):
        a = a + t[:, 8 * c:8 * c + 8]
    for width in (4, 2, 1):
        a = a[:, :width] + a[:, width:]
    return a


def _argmin_body(xT_ref, w_ref, idx_ref, w2_scr):
    @pl.when(pl.program_id(0) == 0)
    def _():
        ww = w_ref[...]
        w2_scr[...] = _norm_over_cols(ww * ww)

    xT = xT_ref[...]
    x2 = _norm_over_rows(xT * xT)
    sT = lax.dot_general(w_ref[...], xT, (((1,), (0,)), ((), ())),
                         preferred_element_type=jnp.float32)
    dT = x2 + w2_scr[...] - 2.0 * sT
    m = jnp.min(dT, axis=0, keepdims=True)
    row = lax.broadcasted_iota(jnp.int32, dT.shape, 0)
    idx_ref[...] = jnp.min(jnp.where(dT == m, row, NUM_Q), axis=0)


def _compute_indices(xT, w):
    grid = N_ROWS // ROW_BLOCK
    return pl.pallas_call(
        _argmin_body,
        grid=(grid,),
        in_specs=[
            pl.BlockSpec((VEC_DIM, ROW_BLOCK), lambda i: (0, i)),
            pl.BlockSpec((NUM_Q, VEC_DIM), lambda i: (0, 0)),
        ],
        out_specs=pl.BlockSpec((ROW_BLOCK,), lambda i: (i,)),
        out_shape=jax.ShapeDtypeStruct((N_ROWS,), jnp.int32),
        scratch_shapes=[pltpu.VMEM((NUM_Q, 1), jnp.float32)],
    )(xT, w)


_GATHER_W = 128


def _gather_rows(table_pad, idx):
    info = plsc.get_sparse_core_info()
    nc, ns = info.num_cores, info.num_subcores
    b_per_w = N_ROWS // (nc * ns)
    mesh = plsc.VectorSubcoreMesh(core_axis_name="c", subcore_axis_name="s")

    def body(table_hbm, idx_hbm, out_hbm, idx_v, rows_v, sem):
        wid = lax.axis_index("s") * nc + lax.axis_index("c")
        base = wid * b_per_w
        pltpu.sync_copy(idx_hbm.at[pl.ds(base, b_per_w)], idx_v)
        pltpu.async_copy(table_hbm.at[idx_v], rows_v, sem).wait()
        pltpu.sync_copy(rows_v, out_hbm.at[pl.ds(base, b_per_w)])

    k = functools.partial(
        pl.kernel,
        mesh=mesh,
        out_type=jax.ShapeDtypeStruct((N_ROWS, VEC_DIM), jnp.float32),
        scratch_types=[
            pltpu.VMEM((b_per_w,), jnp.int32),
            pltpu.VMEM((b_per_w, VEC_DIM), jnp.float32),
            pltpu.SemaphoreType.DMA,
        ],
        compiler_params=pltpu.CompilerParams(use_tc_tiling_on_sc=False),
    )(body)
    return k(table_pad, idx)


def kernel(continuous_vec, W):
    x = continuous_vec.reshape(-1, VEC_DIM).astype(jnp.float32)
    w = W.astype(jnp.float32)
    indices = _compute_indices(x.T, w)
    q = _gather_rows(w, indices)
    return q.reshape(continuous_vec.shape).astype(continuous_vec.dtype)


# speedup vs baseline: 1.0117x; 1.0117x over previous
import functools

import jax
import jax.numpy as jnp
from jax import lax
from jax.experimental import pallas as pl
from jax.experimental.pallas import tpu as pltpu
from jax.experimental.pallas import tpu_sc as plsc

N_ROWS = 16384
NUM_Q = 1024
VEC_DIM = 64
ROW_BLOCK = 8192


def _norm_over_rows(t):
    a = t[0:8, :]
    for c in range(1, 8):
        a = a + t[8 * c:8 * c + 8, :]
    for width in (4, 2, 1):
        a = a[:width, :] + a[width:, :]
    return a


def _norm_over_cols(t):
    a = t[:, 0:8]
    for c in range(1, 8):
        a = a + t[:, 8 * c:8 * c + 8]
    for width in (4, 2, 1):
        a = a[:, :width] + a[:, width:]
    return a


def _argmin_body(xT_ref, w_ref, idx_ref, w2_scr):
    @pl.when(pl.program_id(0) == 0)
    def _():
        ww = w_ref[...]
        w2_scr[...] = _norm_over_cols(ww * ww)

    xT = xT_ref[...]
    x2 = _norm_over_rows(xT * xT)
    sT = lax.dot_general(w_ref[...], xT, (((1,), (0,)), ((), ())),
                         preferred_element_type=jnp.float32)
    dT = x2 + w2_scr[...] - 2.0 * sT
    m = jnp.min(dT, axis=0, keepdims=True)
    row = lax.broadcasted_iota(jnp.int32, dT.shape, 0)
    idx_ref[...] = jnp.min(jnp.where(dT == m, row, NUM_Q), axis=0)


def _compute_indices(xT, w):
    grid = N_ROWS // ROW_BLOCK
    return pl.pallas_call(
        _argmin_body,
        grid=(grid,),
        in_specs=[
            pl.BlockSpec((VEC_DIM, ROW_BLOCK), lambda i: (0, i)),
            pl.BlockSpec((NUM_Q, VEC_DIM), lambda i: (0, 0)),
        ],
        out_specs=pl.BlockSpec((ROW_BLOCK,), lambda i: (i,)),
        out_shape=jax.ShapeDtypeStruct((N_ROWS,), jnp.int32),
        scratch_shapes=[pltpu.VMEM((NUM_Q, 1), jnp.float32)],
    )(xT, w)


_GATHER_W = 128


def _gather_rows(table_pad, idx):
    info = plsc.get_sparse_core_info()
    nc, ns = info.num_cores, info.num_subcores
    b_per_w = N_ROWS // (nc * ns)
    mesh = plsc.VectorSubcoreMesh(core_axis_name="c", subcore_axis_name="s")

    def body(table_hbm, idx_hbm, out_hbm, idx_v, rows_v, sem):
        wid = lax.axis_index("s") * nc + lax.axis_index("c")
        base = wid * b_per_w
        pltpu.sync_copy(idx_hbm.at[pl.ds(base, b_per_w)], idx_v)
        pltpu.async_copy(table_hbm.at[idx_v], rows_v, sem).wait()
        pltpu.sync_copy(rows_v, out_hbm.at[pl.ds(base, b_per_w)])

    k = functools.partial(
        pl.kernel,
        mesh=mesh,
        out_type=jax.ShapeDtypeStruct((N_ROWS, VEC_DIM), jnp.float32),
        scratch_types=[
            pltpu.VMEM((b_per_w,), jnp.int32),
            pltpu.VMEM((b_per_w, VEC_DIM), jnp.float32),
            pltpu.SemaphoreType.DMA,
        ],
        compiler_params=pltpu.CompilerParams(use_tc_tiling_on_sc=False),
    )(body)
    return k(table_pad, idx)


def kernel(continuous_vec, W):
    x = continuous_vec.reshape(-1, VEC_DIM).astype(jnp.float32)
    w = W.astype(jnp.float32)
    indices = _compute_indices(x.T, w)
    q = _gather_rows(w, indices)
    return q.reshape(continuous_vec.shape).astype(continuous_vec.dtype)
